# baseline (speedup 1.0000x reference)
import functools

import jax
import jax.numpy as jnp
from jax import lax
from jax.experimental import pallas as pl
from jax.experimental.pallas import tpu as pltpu
from jax.experimental.pallas import tpu_sc as plsc

G = 128
K = 8


def kernel(x, table):
    B0, B1 = x.shape
    V, D = table.shape
    B = B0 * B1

    info = plsc.get_sparse_core_info()
    NW = info.num_cores * info.num_subcores
    groups_total = B // G
    g_per_w = groups_total // NW
    n_chunks = g_per_w // K
    assert groups_total % NW == 0 and g_per_w % K == 0

    idx2d = x.reshape(groups_total, G).astype(jnp.int32)

    mesh = plsc.VectorSubcoreMesh(core_axis_name="c", subcore_axis_name="s")

    @functools.partial(
        pl.kernel,
        mesh=mesh,
        out_type=jax.ShapeDtypeStruct((groups_total, G, D), jnp.float32),
        scratch_types=[
            pltpu.VMEM((K, G), jnp.int32),
            pltpu.VMEM((K, G, D), jnp.float32),
            pltpu.SemaphoreType.DMA,
        ],
        compiler_params=pltpu.CompilerParams(use_tc_tiling_on_sc=False),
    )
    def emb(idx_hbm, table_hbm, out_hbm, idx_v, rows_v, sem):
        wid = lax.axis_index("s") * info.num_cores + lax.axis_index("c")
        g0 = wid * g_per_w

        def chunk_body(c, carry):
            base = g0 + c * K
            pltpu.sync_copy(idx_hbm.at[pl.ds(base, K)], idx_v)
            for j in range(K):
                pltpu.async_copy(table_hbm.at[idx_v.at[j]], rows_v.at[j], sem)
            for j in range(K):
                pltpu.make_async_copy(
                    table_hbm.at[idx_v.at[j]], rows_v.at[j], sem
                ).wait()
            pltpu.sync_copy(rows_v, out_hbm.at[pl.ds(base, K)])
            return carry

        lax.fori_loop(0, n_chunks, chunk_body, 0)

    out = emb(idx2d, table)
    return out.reshape(B0, B1, D)
